# baseline (speedup 1.0000x reference)
import functools

import jax
import jax.numpy as jnp
from jax import lax
from jax.experimental import pallas as pl
from jax.experimental.pallas import tpu as pltpu
from jax.experimental.pallas import tpu_sc as plsc

_N = 10000
_E = 320000
_H = 128
_C = 64
_NC = 2
_NS = 16
_EPW = _E // _NS
_CHUNK = 128
_NCHUNK = -(-_EPW // _CHUNK)
_EPAD = _NCHUNK * _CHUNK - _EPW
_ACCROWS = _N + 8
_DEPTH = 8
_LOOK = 4
_OWN = 624
_TAIL = _N - _NS * _OWN
_ZTAIL = _ACCROWS - _NS * _OWN
_DEGW = 16
_ZROWS = 24
_ROWBLK = 2000

_HIGH = jax.lax.Precision.HIGHEST


def _make_sc_agg(half: int, with_deg: bool, depth: int, look: int):
    mesh = plsc.VectorSubcoreMesh(core_axis_name="c", subcore_axis_name="s",
                                  num_cores=_NC)
    outs = [jax.ShapeDtypeStruct((_NC, _N, half), jnp.float32)]
    if with_deg:
        outs.append(jax.ShapeDtypeStruct((_NC, _N, _DEGW), jnp.float32))
    scratch = [
        pltpu.VMEM((_NCHUNK, _CHUNK), jnp.int32),
        pltpu.VMEM((_NCHUNK, _CHUNK), jnp.int32),
    ] + [
        pltpu.VMEM((_CHUNK, half), jnp.float32)
        for _ in range(depth)
    ] + [
        pltpu.VMEM((_ZROWS, half), jnp.float32),
        pltpu.VMEM_SHARED((_ACCROWS, half), jnp.float32),
    ] + [pltpu.SemaphoreType.DMA] * (2 * depth)
    if with_deg:
        scratch += [
            pltpu.VMEM((_CHUNK, _DEGW), jnp.float32),
            pltpu.VMEM((_ZROWS, _DEGW), jnp.float32),
            pltpu.VMEM_SHARED((_ACCROWS, _DEGW), jnp.float32),
        ]

    def body(p_hbm, src_hbm, dst_hbm, *rest):
        if with_deg:
            parts_hbm, degparts_hbm = rest[0], rest[1]
            rest = rest[2:]
        else:
            parts_hbm = rest[0]
            rest = rest[1:]
        src_v, dst_v = rest[0], rest[1]
        rows = rest[2:2 + depth]
        zb = rest[2 + depth]
        acc = rest[3 + depth]
        gsem = rest[4 + depth:4 + 2 * depth]
        ssem = rest[4 + 2 * depth:4 + 3 * depth]
        if with_deg:
            ones_v, zb16, dacc = rest[4 + 3 * depth:]
        c = lax.axis_index("c")
        s = lax.axis_index("s")
        base = s * _OWN
        phalf = p_hbm.at[c]

        pltpu.async_copy(src_hbm.at[s], src_v, gsem[0])
        pltpu.async_copy(dst_hbm.at[s], dst_v, gsem[1])

        @pl.loop(0, _ZROWS)
        def _(i):
            @pl.loop(0, half, step=16)
            def _(j):
                zb.at[pl.ds(i, 1), pl.ds(j, 16)][...] = jnp.zeros(
                    (1, 16), jnp.float32)

        @pl.loop(0, _OWN // _ZROWS)
        def _(k):
            pltpu.async_copy(zb, acc.at[pl.ds(base + k * _ZROWS, _ZROWS)],
                             ssem[0])

        @pl.when(s == 0)
        def _():
            pltpu.async_copy(zb.at[pl.ds(0, _ZTAIL)],
                             acc.at[pl.ds(_NS * _OWN, _ZTAIL)], ssem[0])

        if with_deg:
            @pl.loop(0, _CHUNK)
            def _(i):
                ones_v.at[pl.ds(i, 1), pl.ds(0, 16)][...] = jnp.ones(
                    (1, 16), jnp.float32)

            @pl.loop(0, _ZROWS)
            def _(i):
                zb16.at[pl.ds(i, 1), pl.ds(0, 16)][...] = jnp.zeros(
                    (1, 16), jnp.float32)

            @pl.loop(0, _OWN // _ZROWS)
            def _(k):
                pltpu.async_copy(zb16,
                                 dacc.at[pl.ds(base + k * _ZROWS, _ZROWS)],
                                 ssem[1])

            @pl.when(s == 0)
            def _():
                pltpu.async_copy(zb16.at[pl.ds(0, _ZTAIL)],
                                 dacc.at[pl.ds(_NS * _OWN, _ZTAIL)], ssem[1])

        pltpu.make_async_copy(src_hbm.at[s], src_v, gsem[0]).wait()
        pltpu.make_async_copy(dst_hbm.at[s], dst_v, gsem[1]).wait()

        @pl.loop(0, _OWN // _ZROWS)
        def _(k):
            pltpu.make_async_copy(zb, acc.at[pl.ds(base + k * _ZROWS, _ZROWS)],
                                  ssem[0]).wait()

        @pl.when(s == 0)
        def _():
            pltpu.make_async_copy(zb.at[pl.ds(0, _ZTAIL)],
                                  acc.at[pl.ds(_NS * _OWN, _ZTAIL)],
                                  ssem[0]).wait()

        if with_deg:
            @pl.loop(0, _OWN // _ZROWS)
            def _(k):
                pltpu.make_async_copy(
                    zb16, dacc.at[pl.ds(base + k * _ZROWS, _ZROWS)],
                    ssem[1]).wait()

            @pl.when(s == 0)
            def _():
                pltpu.make_async_copy(zb16.at[pl.ds(0, _ZTAIL)],
                                      dacc.at[pl.ds(_NS * _OWN, _ZTAIL)],
                                      ssem[1]).wait()

        plsc.subcore_barrier()

        for b in range(look):
            pltpu.async_copy(phalf.at[src_v.at[b]], rows[b], gsem[b])

        @pl.loop(0, _NCHUNK, step=depth)
        def _(j):
            for k in range(depth):
                m = j + k
                k2 = (k + look) % depth

                @pl.when(m < _NCHUNK)
                def _():
                    pltpu.make_async_copy(phalf.at[src_v.at[m]], rows[k],
                                          gsem[k]).wait()
                    pltpu.async_copy(rows[k], acc.at[dst_v.at[m]], ssem[k],
                                     add=True)

                    @pl.when(m + look < _NCHUNK)
                    def _():
                        @pl.when(m - look >= 0)
                        def _():
                            pltpu.make_async_copy(
                                rows[k2], acc.at[dst_v.at[m - look]],
                                ssem[k2]).wait()

                        pltpu.async_copy(phalf.at[src_v.at[m + look]],
                                         rows[k2], gsem[k2])

        for i in range(depth):
            m = _NCHUNK - depth + i
            pltpu.make_async_copy(rows[m % depth], acc.at[dst_v.at[m]],
                                  ssem[m % depth]).wait()

        if with_deg:
            @pl.loop(0, _NCHUNK)
            def _(j):
                @pl.when(j % _NC == c)
                def _():
                    pltpu.sync_copy(ones_v, dacc.at[dst_v.at[j]], add=True)

        plsc.subcore_barrier()

        pltpu.sync_copy(acc.at[pl.ds(base, _OWN)],
                        parts_hbm.at[c].at[pl.ds(base, _OWN)])

        @pl.when(s == 0)
        def _():
            pltpu.sync_copy(acc.at[pl.ds(_NS * _OWN, _TAIL)],
                            parts_hbm.at[c].at[pl.ds(_NS * _OWN, _TAIL)])

        if with_deg:
            pltpu.sync_copy(dacc.at[pl.ds(base, _OWN)],
                            degparts_hbm.at[c].at[pl.ds(base, _OWN)])

            @pl.when(s == 0)
            def _():
                pltpu.sync_copy(dacc.at[pl.ds(_NS * _OWN, _TAIL)],
                                degparts_hbm.at[c].at[pl.ds(_NS * _OWN, _TAIL)])

    return pl.kernel(
        body,
        out_type=tuple(outs) if with_deg else outs[0],
        mesh=mesh,
        scratch_types=scratch,
        compiler_params=pltpu.CompilerParams(use_tc_tiling_on_sc=False),
    )


@functools.lru_cache(maxsize=None)
def _sc_agg(half: int, with_deg: bool, depth: int, look: int):
    return _make_sc_agg(half, with_deg, depth, look)


def _tc_pre(x, Ws, Wn):
    n, h = x.shape
    wo = Ws.shape[1]
    half = wo // 2

    def kern(x_ref, ws_ref, wn_ref, s_ref, p_ref):
        xb = x_ref[...]
        s_ref[...] = jnp.dot(xb, ws_ref[...], precision=_HIGH,
                             preferred_element_type=jnp.float32)
        p = jnp.dot(xb, wn_ref[...], precision=_HIGH,
                    preferred_element_type=jnp.float32)
        p_ref[0] = p[:, :half]
        p_ref[1] = p[:, half:]

    return pl.pallas_call(
        kern,
        grid=(n // _ROWBLK,),
        in_specs=[
            pl.BlockSpec((_ROWBLK, h), lambda i: (i, 0)),
            pl.BlockSpec((h, wo), lambda i: (0, 0)),
            pl.BlockSpec((h, wo), lambda i: (0, 0)),
        ],
        out_specs=[
            pl.BlockSpec((_ROWBLK, wo), lambda i: (i, 0)),
            pl.BlockSpec((_NC, _ROWBLK, half), lambda i: (0, i, 0)),
        ],
        out_shape=[
            jax.ShapeDtypeStruct((n, wo), jnp.float32),
            jax.ShapeDtypeStruct((_NC, n, half), jnp.float32),
        ],
    )(x, Ws, Wn)


def _combine(parts_ref, deg_ref):
    deg = deg_ref[0, :, 0] + deg_ref[1, :, 0]
    inv = 1.0 / jnp.maximum(deg, 1.0)
    agg = jnp.concatenate([parts_ref[0], parts_ref[1]], axis=-1)
    return agg * inv[:, None]


def _tc_mid(s, parts, degparts, b, g, be, Wsn, Wnn):
    n, h = s.shape
    wo = Wsn.shape[1]
    half = wo // 2
    ph = parts.shape[2]

    def kern(s_ref, parts_ref, deg_ref, b_ref, g_ref, be_ref, ws_ref, wn_ref,
             sn_ref, pn_ref):
        t = s_ref[...] + _combine(parts_ref, deg_ref) + b_ref[...]
        mu = jnp.mean(t, axis=-1, keepdims=True)
        var = jnp.mean((t - mu) ** 2, axis=-1, keepdims=True)
        y = (t - mu) * lax.rsqrt(var + 1e-5) * g_ref[...] + be_ref[...]
        hb = jnp.maximum(y, 0.0)
        sn_ref[...] = jnp.dot(hb, ws_ref[...], precision=_HIGH,
                              preferred_element_type=jnp.float32)
        p = jnp.dot(hb, wn_ref[...], precision=_HIGH,
                    preferred_element_type=jnp.float32)
        pn_ref[0] = p[:, :half]
        pn_ref[1] = p[:, half:]

    return pl.pallas_call(
        kern,
        grid=(n // _ROWBLK,),
        in_specs=[
            pl.BlockSpec((_ROWBLK, h), lambda i: (i, 0)),
            pl.BlockSpec((_NC, _ROWBLK, ph), lambda i: (0, i, 0)),
            pl.BlockSpec((_NC, _ROWBLK, _DEGW), lambda i: (0, i, 0)),
            pl.BlockSpec((1, h), lambda i: (0, 0)),
            pl.BlockSpec((1, h), lambda i: (0, 0)),
            pl.BlockSpec((1, h), lambda i: (0, 0)),
            pl.BlockSpec((h, wo), lambda i: (0, 0)),
            pl.BlockSpec((h, wo), lambda i: (0, 0)),
        ],
        out_specs=[
            pl.BlockSpec((_ROWBLK, wo), lambda i: (i, 0)),
            pl.BlockSpec((_NC, _ROWBLK, half), lambda i: (0, i, 0)),
        ],
        out_shape=[
            jax.ShapeDtypeStruct((n, wo), jnp.float32),
            jax.ShapeDtypeStruct((_NC, n, half), jnp.float32),
        ],
    )(s, parts, degparts, b.reshape(1, h), g.reshape(1, h), be.reshape(1, h),
      Wsn, Wnn)


def _tc_post(s, parts, degparts, b):
    n, w = s.shape
    ph = parts.shape[2]

    def kern(s_ref, parts_ref, deg_ref, b_ref, out_ref):
        out_ref[...] = s_ref[...] + _combine(parts_ref, deg_ref) + b_ref[...]

    return pl.pallas_call(
        kern,
        grid=(n // _ROWBLK,),
        in_specs=[
            pl.BlockSpec((_ROWBLK, w), lambda i: (i, 0)),
            pl.BlockSpec((_NC, _ROWBLK, ph), lambda i: (0, i, 0)),
            pl.BlockSpec((_NC, _ROWBLK, _DEGW), lambda i: (0, i, 0)),
            pl.BlockSpec((1, w), lambda i: (0, 0)),
        ],
        out_specs=pl.BlockSpec((_ROWBLK, w), lambda i: (i, 0)),
        out_shape=jax.ShapeDtypeStruct((n, w), jnp.float32),
    )(s, parts, degparts, b.reshape(1, w))


def kernel(x, edge_index, Ws0, Wn0, b0, g0, be0, Ws1, Wn1, b1, g1, be1,
           Ws2, Wn2, b2):
    src = jnp.pad(edge_index[0].reshape(_NS, _EPW), ((0, 0), (0, _EPAD)),
                  constant_values=0).reshape(_NS, _NCHUNK, _CHUNK)
    dst = jnp.pad(edge_index[1].reshape(_NS, _EPW), ((0, 0), (0, _EPAD)),
                  constant_values=_N).reshape(_NS, _NCHUNK, _CHUNK)

    s0, p0 = _tc_pre(x, Ws0, Wn0)
    parts0, degp = _sc_agg(_H // 2, True, 4, 2)(p0, src, dst)
    s1, p1 = _tc_mid(s0, parts0, degp, b0, g0, be0, Ws1, Wn1)
    parts1 = _sc_agg(_H // 2, False, 6, 3)(p1, src, dst)
    s2, p2 = _tc_mid(s1, parts1, degp, b1, g1, be1, Ws2, Wn2)
    parts2 = _sc_agg(_C // 2, False, 8, 4)(p2, src, dst)
    return _tc_post(s2, parts2, degp, b2)


# speedup vs baseline: 1.1914x; 1.1914x over previous
import functools

import jax
import jax.numpy as jnp
from jax import lax
from jax.experimental import pallas as pl
from jax.experimental.pallas import tpu as pltpu
from jax.experimental.pallas import tpu_sc as plsc

_N = 10000
_E = 320000
_H = 128
_C = 64
_NC = 2
_NS = 16
_EPW = _E // _NS
_CHUNK = 128
_NCHUNK = -(-_EPW // _CHUNK)
_EPAD = _NCHUNK * _CHUNK - _EPW
_ACCROWS = _N + 8
_DEPTH = 8
_LOOK = 4
_OWN = 624
_TAIL = _N - _NS * _OWN
_ZTAIL = _ACCROWS - _NS * _OWN
_DEGW = 16
_ZROWS = 24
_ROWBLK = 2000

_HIGH = jax.lax.Precision.DEFAULT


def _make_sc_agg(half: int, with_deg: bool, depth: int, look: int):
    mesh = plsc.VectorSubcoreMesh(core_axis_name="c", subcore_axis_name="s",
                                  num_cores=_NC)
    outs = [jax.ShapeDtypeStruct((_NC, _N, half), jnp.float32)]
    if with_deg:
        outs.append(jax.ShapeDtypeStruct((_NC, _N, _DEGW), jnp.float32))
    scratch = [
        pltpu.VMEM((_NCHUNK, _CHUNK), jnp.int32),
        pltpu.VMEM((_NCHUNK, _CHUNK), jnp.int32),
    ] + [
        pltpu.VMEM((_CHUNK, half), jnp.float32)
        for _ in range(depth)
    ] + [
        pltpu.VMEM((_ZROWS, half), jnp.float32),
        pltpu.VMEM_SHARED((_ACCROWS, half), jnp.float32),
    ] + [pltpu.SemaphoreType.DMA] * (2 * depth)
    if with_deg:
        scratch += [
            pltpu.VMEM((_CHUNK, _DEGW), jnp.float32),
            pltpu.VMEM((_ZROWS, _DEGW), jnp.float32),
            pltpu.VMEM_SHARED((_ACCROWS, _DEGW), jnp.float32),
        ]

    def body(p_hbm, src_hbm, dst_hbm, *rest):
        if with_deg:
            parts_hbm, degparts_hbm = rest[0], rest[1]
            rest = rest[2:]
        else:
            parts_hbm = rest[0]
            rest = rest[1:]
        src_v, dst_v = rest[0], rest[1]
        rows = rest[2:2 + depth]
        zb = rest[2 + depth]
        acc = rest[3 + depth]
        gsem = rest[4 + depth:4 + 2 * depth]
        ssem = rest[4 + 2 * depth:4 + 3 * depth]
        if with_deg:
            ones_v, zb16, dacc = rest[4 + 3 * depth:]
        c = lax.axis_index("c")
        s = lax.axis_index("s")
        base = s * _OWN
        phalf = p_hbm.at[c]

        pltpu.async_copy(src_hbm.at[s], src_v, gsem[0])
        pltpu.async_copy(dst_hbm.at[s], dst_v, gsem[1])

        @pl.loop(0, _ZROWS)
        def _(i):
            @pl.loop(0, half, step=16)
            def _(j):
                zb.at[pl.ds(i, 1), pl.ds(j, 16)][...] = jnp.zeros(
                    (1, 16), jnp.float32)

        @pl.loop(0, _OWN // _ZROWS)
        def _(k):
            pltpu.async_copy(zb, acc.at[pl.ds(base + k * _ZROWS, _ZROWS)],
                             ssem[0])

        @pl.when(s == 0)
        def _():
            pltpu.async_copy(zb.at[pl.ds(0, _ZTAIL)],
                             acc.at[pl.ds(_NS * _OWN, _ZTAIL)], ssem[0])

        if with_deg:
            @pl.loop(0, _CHUNK)
            def _(i):
                ones_v.at[pl.ds(i, 1), pl.ds(0, 16)][...] = jnp.ones(
                    (1, 16), jnp.float32)

            @pl.loop(0, _ZROWS)
            def _(i):
                zb16.at[pl.ds(i, 1), pl.ds(0, 16)][...] = jnp.zeros(
                    (1, 16), jnp.float32)

            @pl.loop(0, _OWN // _ZROWS)
            def _(k):
                pltpu.async_copy(zb16,
                                 dacc.at[pl.ds(base + k * _ZROWS, _ZROWS)],
                                 ssem[1])

            @pl.when(s == 0)
            def _():
                pltpu.async_copy(zb16.at[pl.ds(0, _ZTAIL)],
                                 dacc.at[pl.ds(_NS * _OWN, _ZTAIL)], ssem[1])

        pltpu.make_async_copy(src_hbm.at[s], src_v, gsem[0]).wait()
        pltpu.make_async_copy(dst_hbm.at[s], dst_v, gsem[1]).wait()

        @pl.loop(0, _OWN // _ZROWS)
        def _(k):
            pltpu.make_async_copy(zb, acc.at[pl.ds(base + k * _ZROWS, _ZROWS)],
                                  ssem[0]).wait()

        @pl.when(s == 0)
        def _():
            pltpu.make_async_copy(zb.at[pl.ds(0, _ZTAIL)],
                                  acc.at[pl.ds(_NS * _OWN, _ZTAIL)],
                                  ssem[0]).wait()

        if with_deg:
            @pl.loop(0, _OWN // _ZROWS)
            def _(k):
                pltpu.make_async_copy(
                    zb16, dacc.at[pl.ds(base + k * _ZROWS, _ZROWS)],
                    ssem[1]).wait()

            @pl.when(s == 0)
            def _():
                pltpu.make_async_copy(zb16.at[pl.ds(0, _ZTAIL)],
                                      dacc.at[pl.ds(_NS * _OWN, _ZTAIL)],
                                      ssem[1]).wait()

        plsc.subcore_barrier()

        for b in range(look):
            pltpu.async_copy(phalf.at[src_v.at[b]], rows[b], gsem[b])

        @pl.loop(0, _NCHUNK, step=depth)
        def _(j):
            for k in range(depth):
                m = j + k
                k2 = (k + look) % depth

                @pl.when(m < _NCHUNK)
                def _():
                    pltpu.make_async_copy(phalf.at[src_v.at[m]], rows[k],
                                          gsem[k]).wait()
                    pltpu.async_copy(rows[k], acc.at[dst_v.at[m]], ssem[k],
                                     add=True)

                    @pl.when(m + look < _NCHUNK)
                    def _():
                        @pl.when(m - look >= 0)
                        def _():
                            pltpu.make_async_copy(
                                rows[k2], acc.at[dst_v.at[m - look]],
                                ssem[k2]).wait()

                        pltpu.async_copy(phalf.at[src_v.at[m + look]],
                                         rows[k2], gsem[k2])

        for i in range(depth):
            m = _NCHUNK - depth + i
            pltpu.make_async_copy(rows[m % depth], acc.at[dst_v.at[m]],
                                  ssem[m % depth]).wait()

        if with_deg:
            @pl.loop(0, _NCHUNK)
            def _(j):
                @pl.when(j % _NC == c)
                def _():
                    pltpu.sync_copy(ones_v, dacc.at[dst_v.at[j]], add=True)

        plsc.subcore_barrier()

        pltpu.sync_copy(acc.at[pl.ds(base, _OWN)],
                        parts_hbm.at[c].at[pl.ds(base, _OWN)])

        @pl.when(s == 0)
        def _():
            pltpu.sync_copy(acc.at[pl.ds(_NS * _OWN, _TAIL)],
                            parts_hbm.at[c].at[pl.ds(_NS * _OWN, _TAIL)])

        if with_deg:
            pltpu.sync_copy(dacc.at[pl.ds(base, _OWN)],
                            degparts_hbm.at[c].at[pl.ds(base, _OWN)])

            @pl.when(s == 0)
            def _():
                pltpu.sync_copy(dacc.at[pl.ds(_NS * _OWN, _TAIL)],
                                degparts_hbm.at[c].at[pl.ds(_NS * _OWN, _TAIL)])

    return pl.kernel(
        body,
        out_type=tuple(outs) if with_deg else outs[0],
        mesh=mesh,
        scratch_types=scratch,
        compiler_params=pltpu.CompilerParams(use_tc_tiling_on_sc=False),
    )


@functools.lru_cache(maxsize=None)
def _sc_agg(half: int, with_deg: bool, depth: int, look: int):
    return _make_sc_agg(half, with_deg, depth, look)


def _tc_pre(x, Ws, Wn):
    n, h = x.shape
    wo = Ws.shape[1]
    half = wo // 2

    def kern(x_ref, ws_ref, wn_ref, s_ref, p_ref):
        xb = x_ref[...]
        s_ref[...] = jnp.dot(xb, ws_ref[...], precision=_HIGH,
                             preferred_element_type=jnp.float32)
        p = jnp.dot(xb, wn_ref[...], precision=_HIGH,
                    preferred_element_type=jnp.float32)
        p_ref[0] = p[:, :half]
        p_ref[1] = p[:, half:]

    return pl.pallas_call(
        kern,
        grid=(n // _ROWBLK,),
        in_specs=[
            pl.BlockSpec((_ROWBLK, h), lambda i: (i, 0)),
            pl.BlockSpec((h, wo), lambda i: (0, 0)),
            pl.BlockSpec((h, wo), lambda i: (0, 0)),
        ],
        out_specs=[
            pl.BlockSpec((_ROWBLK, wo), lambda i: (i, 0)),
            pl.BlockSpec((_NC, _ROWBLK, half), lambda i: (0, i, 0)),
        ],
        out_shape=[
            jax.ShapeDtypeStruct((n, wo), jnp.float32),
            jax.ShapeDtypeStruct((_NC, n, half), jnp.float32),
        ],
    )(x, Ws, Wn)


def _combine(parts_ref, deg_ref):
    deg = deg_ref[0, :, 0] + deg_ref[1, :, 0]
    inv = 1.0 / jnp.maximum(deg, 1.0)
    agg = jnp.concatenate([parts_ref[0], parts_ref[1]], axis=-1)
    return agg * inv[:, None]


def _tc_mid(s, parts, degparts, b, g, be, Wsn, Wnn):
    n, h = s.shape
    wo = Wsn.shape[1]
    half = wo // 2
    ph = parts.shape[2]

    def kern(s_ref, parts_ref, deg_ref, b_ref, g_ref, be_ref, ws_ref, wn_ref,
             sn_ref, pn_ref):
        t = s_ref[...] + _combine(parts_ref, deg_ref) + b_ref[...]
        mu = jnp.mean(t, axis=-1, keepdims=True)
        var = jnp.mean((t - mu) ** 2, axis=-1, keepdims=True)
        y = (t - mu) * lax.rsqrt(var + 1e-5) * g_ref[...] + be_ref[...]
        hb = jnp.maximum(y, 0.0)
        sn_ref[...] = jnp.dot(hb, ws_ref[...], precision=_HIGH,
                              preferred_element_type=jnp.float32)
        p = jnp.dot(hb, wn_ref[...], precision=_HIGH,
                    preferred_element_type=jnp.float32)
        pn_ref[0] = p[:, :half]
        pn_ref[1] = p[:, half:]

    return pl.pallas_call(
        kern,
        grid=(n // _ROWBLK,),
        in_specs=[
            pl.BlockSpec((_ROWBLK, h), lambda i: (i, 0)),
            pl.BlockSpec((_NC, _ROWBLK, ph), lambda i: (0, i, 0)),
            pl.BlockSpec((_NC, _ROWBLK, _DEGW), lambda i: (0, i, 0)),
            pl.BlockSpec((1, h), lambda i: (0, 0)),
            pl.BlockSpec((1, h), lambda i: (0, 0)),
            pl.BlockSpec((1, h), lambda i: (0, 0)),
            pl.BlockSpec((h, wo), lambda i: (0, 0)),
            pl.BlockSpec((h, wo), lambda i: (0, 0)),
        ],
        out_specs=[
            pl.BlockSpec((_ROWBLK, wo), lambda i: (i, 0)),
            pl.BlockSpec((_NC, _ROWBLK, half), lambda i: (0, i, 0)),
        ],
        out_shape=[
            jax.ShapeDtypeStruct((n, wo), jnp.float32),
            jax.ShapeDtypeStruct((_NC, n, half), jnp.float32),
        ],
    )(s, parts, degparts, b.reshape(1, h), g.reshape(1, h), be.reshape(1, h),
      Wsn, Wnn)


def _tc_post(s, parts, degparts, b):
    n, w = s.shape
    ph = parts.shape[2]

    def kern(s_ref, parts_ref, deg_ref, b_ref, out_ref):
        out_ref[...] = s_ref[...] + _combine(parts_ref, deg_ref) + b_ref[...]

    return pl.pallas_call(
        kern,
        grid=(n // _ROWBLK,),
        in_specs=[
            pl.BlockSpec((_ROWBLK, w), lambda i: (i, 0)),
            pl.BlockSpec((_NC, _ROWBLK, ph), lambda i: (0, i, 0)),
            pl.BlockSpec((_NC, _ROWBLK, _DEGW), lambda i: (0, i, 0)),
            pl.BlockSpec((1, w), lambda i: (0, 0)),
        ],
        out_specs=pl.BlockSpec((_ROWBLK, w), lambda i: (i, 0)),
        out_shape=jax.ShapeDtypeStruct((n, w), jnp.float32),
    )(s, parts, degparts, b.reshape(1, w))


def kernel(x, edge_index, Ws0, Wn0, b0, g0, be0, Ws1, Wn1, b1, g1, be1,
           Ws2, Wn2, b2):
    src = jnp.pad(edge_index[0].reshape(_NS, _EPW), ((0, 0), (0, _EPAD)),
                  constant_values=0).reshape(_NS, _NCHUNK, _CHUNK)
    dst = jnp.pad(edge_index[1].reshape(_NS, _EPW), ((0, 0), (0, _EPAD)),
                  constant_values=_N).reshape(_NS, _NCHUNK, _CHUNK)

    s0, p0 = _tc_pre(x, Ws0, Wn0)
    parts0, degp = _sc_agg(_H // 2, True, 4, 2)(p0, src, dst)
    s1, p1 = _tc_mid(s0, parts0, degp, b0, g0, be0, Ws1, Wn1)
    parts1 = _sc_agg(_H // 2, False, 6, 3)(p1, src, dst)
    s2, p2 = _tc_mid(s1, parts1, degp, b1, g1, be1, Ws2, Wn2)
    parts2 = _sc_agg(_C // 2, False, 8, 4)(p2, src, dst)
    return _tc_post(s2, parts2, degp, b2)


# speedup vs baseline: 1.2098x; 1.0154x over previous
import functools

import jax
import jax.numpy as jnp
from jax import lax
from jax.experimental import pallas as pl
from jax.experimental.pallas import tpu as pltpu
from jax.experimental.pallas import tpu_sc as plsc

_N = 10000
_E = 320000
_H = 128
_C = 64
_NC = 2
_NS = 16
_EPW = _E // _NS
_CHUNK = 128
_NCHUNK = -(-_EPW // _CHUNK)
_EPAD = _NCHUNK * _CHUNK - _EPW
_ACCROWS = _N + 8
_DEPTH = 8
_LOOK = 4
_OWN = 624
_TAIL = _N - _NS * _OWN
_ZTAIL = _ACCROWS - _NS * _OWN
_DEGW = 16
_ZROWS = 24
_ROWBLK = 2000

_HIGH = jax.lax.Precision.DEFAULT


def _make_sc_agg(half: int, with_deg: bool, depth: int, look: int):
    mesh = plsc.VectorSubcoreMesh(core_axis_name="c", subcore_axis_name="s",
                                  num_cores=_NC)
    outs = [jax.ShapeDtypeStruct((_NC, _N, half), jnp.float32)]
    if with_deg:
        outs.append(jax.ShapeDtypeStruct((_NC, _N, _DEGW), jnp.float32))
    scratch = [
        pltpu.VMEM((_NCHUNK, _CHUNK), jnp.int32),
        pltpu.VMEM((_NCHUNK, _CHUNK), jnp.int32),
    ] + [
        pltpu.VMEM((_CHUNK, half), jnp.float32)
        for _ in range(depth)
    ] + [
        pltpu.VMEM((_ZROWS, half), jnp.float32),
        pltpu.VMEM_SHARED((_ACCROWS, half), jnp.float32),
    ] + [pltpu.SemaphoreType.DMA] * (2 * depth)
    if with_deg:
        scratch += [
            pltpu.VMEM((_CHUNK, _DEGW), jnp.float32),
            pltpu.VMEM((_ZROWS, _DEGW), jnp.float32),
            pltpu.VMEM_SHARED((_ACCROWS, _DEGW), jnp.float32),
        ]

    def body(p_hbm, src_hbm, dst_hbm, *rest):
        if with_deg:
            parts_hbm, degparts_hbm = rest[0], rest[1]
            rest = rest[2:]
        else:
            parts_hbm = rest[0]
            rest = rest[1:]
        src_v, dst_v = rest[0], rest[1]
        rows = rest[2:2 + depth]
        zb = rest[2 + depth]
        acc = rest[3 + depth]
        gsem = rest[4 + depth:4 + 2 * depth]
        ssem = rest[4 + 2 * depth:4 + 3 * depth]
        if with_deg:
            ones_v, zb16, dacc = rest[4 + 3 * depth:]
        c = lax.axis_index("c")
        s = lax.axis_index("s")
        base = s * _OWN
        phalf = p_hbm.at[c]

        pltpu.async_copy(src_hbm.at[s], src_v, gsem[0])
        pltpu.async_copy(dst_hbm.at[s], dst_v, gsem[1])

        @pl.loop(0, _ZROWS)
        def _(i):
            @pl.loop(0, half, step=16)
            def _(j):
                zb.at[pl.ds(i, 1), pl.ds(j, 16)][...] = jnp.zeros(
                    (1, 16), jnp.float32)

        @pl.loop(0, _OWN // _ZROWS)
        def _(k):
            pltpu.async_copy(zb, acc.at[pl.ds(base + k * _ZROWS, _ZROWS)],
                             ssem[0])

        @pl.when(s == 0)
        def _():
            pltpu.async_copy(zb.at[pl.ds(0, _ZTAIL)],
                             acc.at[pl.ds(_NS * _OWN, _ZTAIL)], ssem[0])

        if with_deg:
            @pl.loop(0, _CHUNK)
            def _(i):
                ones_v.at[pl.ds(i, 1), pl.ds(0, 16)][...] = jnp.ones(
                    (1, 16), jnp.float32)

            @pl.loop(0, _ZROWS)
            def _(i):
                zb16.at[pl.ds(i, 1), pl.ds(0, 16)][...] = jnp.zeros(
                    (1, 16), jnp.float32)

            @pl.loop(0, _OWN // _ZROWS)
            def _(k):
                pltpu.async_copy(zb16,
                                 dacc.at[pl.ds(base + k * _ZROWS, _ZROWS)],
                                 ssem[1])

            @pl.when(s == 0)
            def _():
                pltpu.async_copy(zb16.at[pl.ds(0, _ZTAIL)],
                                 dacc.at[pl.ds(_NS * _OWN, _ZTAIL)], ssem[1])

        pltpu.make_async_copy(src_hbm.at[s], src_v, gsem[0]).wait()
        pltpu.make_async_copy(dst_hbm.at[s], dst_v, gsem[1]).wait()

        @pl.loop(0, _OWN // _ZROWS)
        def _(k):
            pltpu.make_async_copy(zb, acc.at[pl.ds(base + k * _ZROWS, _ZROWS)],
                                  ssem[0]).wait()

        @pl.when(s == 0)
        def _():
            pltpu.make_async_copy(zb.at[pl.ds(0, _ZTAIL)],
                                  acc.at[pl.ds(_NS * _OWN, _ZTAIL)],
                                  ssem[0]).wait()

        if with_deg:
            @pl.loop(0, _OWN // _ZROWS)
            def _(k):
                pltpu.make_async_copy(
                    zb16, dacc.at[pl.ds(base + k * _ZROWS, _ZROWS)],
                    ssem[1]).wait()

            @pl.when(s == 0)
            def _():
                pltpu.make_async_copy(zb16.at[pl.ds(0, _ZTAIL)],
                                      dacc.at[pl.ds(_NS * _OWN, _ZTAIL)],
                                      ssem[1]).wait()

        plsc.subcore_barrier()

        for b in range(look):
            pltpu.async_copy(phalf.at[src_v.at[b]], rows[b], gsem[b])

        @pl.loop(0, _NCHUNK, step=depth)
        def _(j):
            for k in range(depth):
                m = j + k
                k2 = (k + look) % depth

                @pl.when(m < _NCHUNK)
                def _():
                    pltpu.make_async_copy(phalf.at[src_v.at[m]], rows[k],
                                          gsem[k]).wait()
                    pltpu.async_copy(rows[k], acc.at[dst_v.at[m]], ssem[k],
                                     add=True)
                    if with_deg:
                        @pl.when(m % _NC == c)
                        def _():
                            pltpu.async_copy(ones_v, dacc.at[dst_v.at[m]],
                                             ssem[k], add=True)

                    @pl.when(m + look < _NCHUNK)
                    def _():
                        @pl.when(m - look >= 0)
                        def _():
                            pltpu.make_async_copy(
                                rows[k2], acc.at[dst_v.at[m - look]],
                                ssem[k2]).wait()
                            if with_deg:
                                @pl.when((m - look) % _NC == c)
                                def _():
                                    pltpu.make_async_copy(
                                        ones_v, dacc.at[dst_v.at[m - look]],
                                        ssem[k2]).wait()

                        pltpu.async_copy(phalf.at[src_v.at[m + look]],
                                         rows[k2], gsem[k2])

        for i in range(depth):
            m = _NCHUNK - depth + i
            pltpu.make_async_copy(rows[m % depth], acc.at[dst_v.at[m]],
                                  ssem[m % depth]).wait()
            if with_deg:
                @pl.when(m % _NC == c)
                def _():
                    pltpu.make_async_copy(ones_v, dacc.at[dst_v.at[m]],
                                          ssem[m % depth]).wait()

        plsc.subcore_barrier()

        pltpu.sync_copy(acc.at[pl.ds(base, _OWN)],
                        parts_hbm.at[c].at[pl.ds(base, _OWN)])

        @pl.when(s == 0)
        def _():
            pltpu.sync_copy(acc.at[pl.ds(_NS * _OWN, _TAIL)],
                            parts_hbm.at[c].at[pl.ds(_NS * _OWN, _TAIL)])

        if with_deg:
            pltpu.sync_copy(dacc.at[pl.ds(base, _OWN)],
                            degparts_hbm.at[c].at[pl.ds(base, _OWN)])

            @pl.when(s == 0)
            def _():
                pltpu.sync_copy(dacc.at[pl.ds(_NS * _OWN, _TAIL)],
                                degparts_hbm.at[c].at[pl.ds(_NS * _OWN, _TAIL)])

    return pl.kernel(
        body,
        out_type=tuple(outs) if with_deg else outs[0],
        mesh=mesh,
        scratch_types=scratch,
        compiler_params=pltpu.CompilerParams(use_tc_tiling_on_sc=False),
    )


@functools.lru_cache(maxsize=None)
def _sc_agg(half: int, with_deg: bool, depth: int, look: int):
    return _make_sc_agg(half, with_deg, depth, look)


def _tc_pre(x, Ws, Wn):
    n, h = x.shape
    wo = Ws.shape[1]
    half = wo // 2

    def kern(x_ref, ws_ref, wn_ref, s_ref, p_ref):
        xb = x_ref[...]
        s_ref[...] = jnp.dot(xb, ws_ref[...], precision=_HIGH,
                             preferred_element_type=jnp.float32)
        p = jnp.dot(xb, wn_ref[...], precision=_HIGH,
                    preferred_element_type=jnp.float32)
        p_ref[0] = p[:, :half]
        p_ref[1] = p[:, half:]

    return pl.pallas_call(
        kern,
        grid=(n // _ROWBLK,),
        in_specs=[
            pl.BlockSpec((_ROWBLK, h), lambda i: (i, 0)),
            pl.BlockSpec((h, wo), lambda i: (0, 0)),
            pl.BlockSpec((h, wo), lambda i: (0, 0)),
        ],
        out_specs=[
            pl.BlockSpec((_ROWBLK, wo), lambda i: (i, 0)),
            pl.BlockSpec((_NC, _ROWBLK, half), lambda i: (0, i, 0)),
        ],
        out_shape=[
            jax.ShapeDtypeStruct((n, wo), jnp.float32),
            jax.ShapeDtypeStruct((_NC, n, half), jnp.float32),
        ],
    )(x, Ws, Wn)


def _combine(parts_ref, deg_ref):
    deg = deg_ref[0, :, 0] + deg_ref[1, :, 0]
    inv = 1.0 / jnp.maximum(deg, 1.0)
    agg = jnp.concatenate([parts_ref[0], parts_ref[1]], axis=-1)
    return agg * inv[:, None]


def _tc_mid(s, parts, degparts, b, g, be, Wsn, Wnn):
    n, h = s.shape
    wo = Wsn.shape[1]
    half = wo // 2
    ph = parts.shape[2]

    def kern(s_ref, parts_ref, deg_ref, b_ref, g_ref, be_ref, ws_ref, wn_ref,
             sn_ref, pn_ref):
        t = s_ref[...] + _combine(parts_ref, deg_ref) + b_ref[...]
        mu = jnp.mean(t, axis=-1, keepdims=True)
        var = jnp.mean((t - mu) ** 2, axis=-1, keepdims=True)
        y = (t - mu) * lax.rsqrt(var + 1e-5) * g_ref[...] + be_ref[...]
        hb = jnp.maximum(y, 0.0)
        sn_ref[...] = jnp.dot(hb, ws_ref[...], precision=_HIGH,
                              preferred_element_type=jnp.float32)
        p = jnp.dot(hb, wn_ref[...], precision=_HIGH,
                    preferred_element_type=jnp.float32)
        pn_ref[0] = p[:, :half]
        pn_ref[1] = p[:, half:]

    return pl.pallas_call(
        kern,
        grid=(n // _ROWBLK,),
        in_specs=[
            pl.BlockSpec((_ROWBLK, h), lambda i: (i, 0)),
            pl.BlockSpec((_NC, _ROWBLK, ph), lambda i: (0, i, 0)),
            pl.BlockSpec((_NC, _ROWBLK, _DEGW), lambda i: (0, i, 0)),
            pl.BlockSpec((1, h), lambda i: (0, 0)),
            pl.BlockSpec((1, h), lambda i: (0, 0)),
            pl.BlockSpec((1, h), lambda i: (0, 0)),
            pl.BlockSpec((h, wo), lambda i: (0, 0)),
            pl.BlockSpec((h, wo), lambda i: (0, 0)),
        ],
        out_specs=[
            pl.BlockSpec((_ROWBLK, wo), lambda i: (i, 0)),
            pl.BlockSpec((_NC, _ROWBLK, half), lambda i: (0, i, 0)),
        ],
        out_shape=[
            jax.ShapeDtypeStruct((n, wo), jnp.float32),
            jax.ShapeDtypeStruct((_NC, n, half), jnp.float32),
        ],
    )(s, parts, degparts, b.reshape(1, h), g.reshape(1, h), be.reshape(1, h),
      Wsn, Wnn)


def _tc_post(s, parts, degparts, b):
    n, w = s.shape
    ph = parts.shape[2]

    def kern(s_ref, parts_ref, deg_ref, b_ref, out_ref):
        out_ref[...] = s_ref[...] + _combine(parts_ref, deg_ref) + b_ref[...]

    return pl.pallas_call(
        kern,
        grid=(n // _ROWBLK,),
        in_specs=[
            pl.BlockSpec((_ROWBLK, w), lambda i: (i, 0)),
            pl.BlockSpec((_NC, _ROWBLK, ph), lambda i: (0, i, 0)),
            pl.BlockSpec((_NC, _ROWBLK, _DEGW), lambda i: (0, i, 0)),
            pl.BlockSpec((1, w), lambda i: (0, 0)),
        ],
        out_specs=pl.BlockSpec((_ROWBLK, w), lambda i: (i, 0)),
        out_shape=jax.ShapeDtypeStruct((n, w), jnp.float32),
    )(s, parts, degparts, b.reshape(1, w))


def kernel(x, edge_index, Ws0, Wn0, b0, g0, be0, Ws1, Wn1, b1, g1, be1,
           Ws2, Wn2, b2):
    src = jnp.pad(edge_index[0].reshape(_NS, _EPW), ((0, 0), (0, _EPAD)),
                  constant_values=0).reshape(_NS, _NCHUNK, _CHUNK)
    dst = jnp.pad(edge_index[1].reshape(_NS, _EPW), ((0, 0), (0, _EPAD)),
                  constant_values=_N).reshape(_NS, _NCHUNK, _CHUNK)

    s0, p0 = _tc_pre(x, Ws0, Wn0)
    parts0, degp = _sc_agg(_H // 2, True, 4, 2)(p0, src, dst)
    s1, p1 = _tc_mid(s0, parts0, degp, b0, g0, be0, Ws1, Wn1)
    parts1 = _sc_agg(_H // 2, False, 6, 3)(p1, src, dst)
    s2, p2 = _tc_mid(s1, parts1, degp, b1, g1, be1, Ws2, Wn2)
    parts2 = _sc_agg(_C // 2, False, 8, 4)(p2, src, dst)
    return _tc_post(s2, parts2, degp, b2)


# speedup vs baseline: 1.3132x; 1.0854x over previous
import functools

import jax
import jax.numpy as jnp
from jax import lax
from jax.experimental import pallas as pl
from jax.experimental.pallas import tpu as pltpu
from jax.experimental.pallas import tpu_sc as plsc

_N = 10000
_E = 320000
_H = 128
_C = 64
_NC = 2
_NS = 16
_EPW = _E // _NS
_CHUNK = 80
_NCHUNK = _EPW // _CHUNK
_ACCROWS = _N + 8
_DEPTH = 8
_LOOK = 4
_OWN = 624
_TAIL = _N - _NS * _OWN
_ZTAIL = _ACCROWS - _NS * _OWN
_DEGW = 16
_ZROWS = 24
_ROWBLK = 2000

_HIGH = jax.lax.Precision.DEFAULT


def _make_sc_agg(half: int, with_deg: bool, depth: int, look: int):
    mesh = plsc.VectorSubcoreMesh(core_axis_name="c", subcore_axis_name="s",
                                  num_cores=_NC)
    outs = [jax.ShapeDtypeStruct((_NC, _N, half), jnp.float32)]
    if with_deg:
        outs.append(jax.ShapeDtypeStruct((_NC, _N, _DEGW), jnp.float32))
    scratch = [
        pltpu.VMEM((_NCHUNK, _CHUNK), jnp.int32),
        pltpu.VMEM((_NCHUNK, _CHUNK), jnp.int32),
    ] + [
        pltpu.VMEM((_CHUNK, half), jnp.float32)
        for _ in range(depth)
    ] + [
        pltpu.VMEM((_ZROWS, half), jnp.float32),
        pltpu.VMEM_SHARED((_ACCROWS, half), jnp.float32),
    ] + [pltpu.SemaphoreType.DMA] * (2 * depth)
    if with_deg:
        scratch += [
            pltpu.VMEM((_CHUNK, _DEGW), jnp.float32),
            pltpu.VMEM((_ZROWS, _DEGW), jnp.float32),
            pltpu.VMEM_SHARED((_ACCROWS, _DEGW), jnp.float32),
        ]

    def body(p_hbm, src_hbm, dst_hbm, *rest):
        if with_deg:
            parts_hbm, degparts_hbm = rest[0], rest[1]
            rest = rest[2:]
        else:
            parts_hbm = rest[0]
            rest = rest[1:]
        src_v, dst_v = rest[0], rest[1]
        rows = rest[2:2 + depth]
        zb = rest[2 + depth]
        acc = rest[3 + depth]
        gsem = rest[4 + depth:4 + 2 * depth]
        ssem = rest[4 + 2 * depth:4 + 3 * depth]
        if with_deg:
            ones_v, zb16, dacc = rest[4 + 3 * depth:]
        c = lax.axis_index("c")
        s = lax.axis_index("s")
        base = s * _OWN
        phalf = p_hbm.at[c]

        pltpu.async_copy(src_hbm.at[s], src_v, gsem[0])
        pltpu.async_copy(dst_hbm.at[s], dst_v, gsem[1])

        @pl.loop(0, _ZROWS)
        def _(i):
            @pl.loop(0, half, step=16)
            def _(j):
                zb.at[pl.ds(i, 1), pl.ds(j, 16)][...] = jnp.zeros(
                    (1, 16), jnp.float32)

        @pl.loop(0, _OWN // _ZROWS)
        def _(k):
            pltpu.async_copy(zb, acc.at[pl.ds(base + k * _ZROWS, _ZROWS)],
                             ssem[0])

        @pl.when(s == 0)
        def _():
            pltpu.async_copy(zb.at[pl.ds(0, _ZTAIL)],
                             acc.at[pl.ds(_NS * _OWN, _ZTAIL)], ssem[0])

        if with_deg:
            @pl.loop(0, _CHUNK)
            def _(i):
                ones_v.at[pl.ds(i, 1), pl.ds(0, 16)][...] = jnp.ones(
                    (1, 16), jnp.float32)

            @pl.loop(0, _ZROWS)
            def _(i):
                zb16.at[pl.ds(i, 1), pl.ds(0, 16)][...] = jnp.zeros(
                    (1, 16), jnp.float32)

            @pl.loop(0, _OWN // _ZROWS)
            def _(k):
                pltpu.async_copy(zb16,
                                 dacc.at[pl.ds(base + k * _ZROWS, _ZROWS)],
                                 ssem[1])

            @pl.when(s == 0)
            def _():
                pltpu.async_copy(zb16.at[pl.ds(0, _ZTAIL)],
                                 dacc.at[pl.ds(_NS * _OWN, _ZTAIL)], ssem[1])

        pltpu.make_async_copy(src_hbm.at[s], src_v, gsem[0]).wait()
        pltpu.make_async_copy(dst_hbm.at[s], dst_v, gsem[1]).wait()

        @pl.loop(0, _OWN // _ZROWS)
        def _(k):
            pltpu.make_async_copy(zb, acc.at[pl.ds(base + k * _ZROWS, _ZROWS)],
                                  ssem[0]).wait()

        @pl.when(s == 0)
        def _():
            pltpu.make_async_copy(zb.at[pl.ds(0, _ZTAIL)],
                                  acc.at[pl.ds(_NS * _OWN, _ZTAIL)],
                                  ssem[0]).wait()

        if with_deg:
            @pl.loop(0, _OWN // _ZROWS)
            def _(k):
                pltpu.make_async_copy(
                    zb16, dacc.at[pl.ds(base + k * _ZROWS, _ZROWS)],
                    ssem[1]).wait()

            @pl.when(s == 0)
            def _():
                pltpu.make_async_copy(zb16.at[pl.ds(0, _ZTAIL)],
                                      dacc.at[pl.ds(_NS * _OWN, _ZTAIL)],
                                      ssem[1]).wait()

        plsc.subcore_barrier()

        for b in range(look):
            pltpu.async_copy(phalf.at[src_v.at[b]], rows[b], gsem[b])

        @pl.loop(0, _NCHUNK, step=depth)
        def _(j):
            for k in range(depth):
                m = j + k
                k2 = (k + look) % depth

                @pl.when(m < _NCHUNK)
                def _():
                    pltpu.make_async_copy(phalf.at[src_v.at[m]], rows[k],
                                          gsem[k]).wait()
                    pltpu.async_copy(rows[k], acc.at[dst_v.at[m]], ssem[k],
                                     add=True)
                    if with_deg:
                        @pl.when(m % _NC == c)
                        def _():
                            pltpu.async_copy(ones_v, dacc.at[dst_v.at[m]],
                                             ssem[k], add=True)

                    @pl.when(m + look < _NCHUNK)
                    def _():
                        @pl.when(m - look >= 0)
                        def _():
                            pltpu.make_async_copy(
                                rows[k2], acc.at[dst_v.at[m - look]],
                                ssem[k2]).wait()
                            if with_deg:
                                @pl.when((m - look) % _NC == c)
                                def _():
                                    pltpu.make_async_copy(
                                        ones_v, dacc.at[dst_v.at[m - look]],
                                        ssem[k2]).wait()

                        pltpu.async_copy(phalf.at[src_v.at[m + look]],
                                         rows[k2], gsem[k2])

        for i in range(depth):
            m = _NCHUNK - depth + i
            pltpu.make_async_copy(rows[m % depth], acc.at[dst_v.at[m]],
                                  ssem[m % depth]).wait()
            if with_deg:
                @pl.when(m % _NC == c)
                def _():
                    pltpu.make_async_copy(ones_v, dacc.at[dst_v.at[m]],
                                          ssem[m % depth]).wait()

        plsc.subcore_barrier()

        pltpu.sync_copy(acc.at[pl.ds(base, _OWN)],
                        parts_hbm.at[c].at[pl.ds(base, _OWN)])

        @pl.when(s == 0)
        def _():
            pltpu.sync_copy(acc.at[pl.ds(_NS * _OWN, _TAIL)],
                            parts_hbm.at[c].at[pl.ds(_NS * _OWN, _TAIL)])

        if with_deg:
            pltpu.sync_copy(dacc.at[pl.ds(base, _OWN)],
                            degparts_hbm.at[c].at[pl.ds(base, _OWN)])

            @pl.when(s == 0)
            def _():
                pltpu.sync_copy(dacc.at[pl.ds(_NS * _OWN, _TAIL)],
                                degparts_hbm.at[c].at[pl.ds(_NS * _OWN, _TAIL)])

    return pl.kernel(
        body,
        out_type=tuple(outs) if with_deg else outs[0],
        mesh=mesh,
        scratch_types=scratch,
        compiler_params=pltpu.CompilerParams(use_tc_tiling_on_sc=False),
    )


@functools.lru_cache(maxsize=None)
def _sc_agg(half: int, with_deg: bool, depth: int, look: int):
    return _make_sc_agg(half, with_deg, depth, look)


def _tc_pre(x, Ws, Wn):
    n, h = x.shape
    wo = Ws.shape[1]
    half = wo // 2

    def kern(x_ref, ws_ref, wn_ref, s_ref, p_ref):
        xb = x_ref[...]
        s_ref[...] = jnp.dot(xb, ws_ref[...], precision=_HIGH,
                             preferred_element_type=jnp.float32)
        p = jnp.dot(xb, wn_ref[...], precision=_HIGH,
                    preferred_element_type=jnp.float32)
        p_ref[0] = p[:, :half]
        p_ref[1] = p[:, half:]

    return pl.pallas_call(
        kern,
        grid=(n // _ROWBLK,),
        in_specs=[
            pl.BlockSpec((_ROWBLK, h), lambda i: (i, 0)),
            pl.BlockSpec((h, wo), lambda i: (0, 0)),
            pl.BlockSpec((h, wo), lambda i: (0, 0)),
        ],
        out_specs=[
            pl.BlockSpec((_ROWBLK, wo), lambda i: (i, 0)),
            pl.BlockSpec((_NC, _ROWBLK, half), lambda i: (0, i, 0)),
        ],
        out_shape=[
            jax.ShapeDtypeStruct((n, wo), jnp.float32),
            jax.ShapeDtypeStruct((_NC, n, half), jnp.float32),
        ],
    )(x, Ws, Wn)


def _combine(parts_ref, deg_ref):
    deg = deg_ref[0, :, 0] + deg_ref[1, :, 0]
    inv = 1.0 / jnp.maximum(deg, 1.0)
    agg = jnp.concatenate([parts_ref[0], parts_ref[1]], axis=-1)
    return agg * inv[:, None]


def _tc_mid(s, parts, degparts, b, g, be, Wsn, Wnn):
    n, h = s.shape
    wo = Wsn.shape[1]
    half = wo // 2
    ph = parts.shape[2]

    def kern(s_ref, parts_ref, deg_ref, b_ref, g_ref, be_ref, ws_ref, wn_ref,
             sn_ref, pn_ref):
        t = s_ref[...] + _combine(parts_ref, deg_ref) + b_ref[...]
        mu = jnp.mean(t, axis=-1, keepdims=True)
        var = jnp.mean((t - mu) ** 2, axis=-1, keepdims=True)
        y = (t - mu) * lax.rsqrt(var + 1e-5) * g_ref[...] + be_ref[...]
        hb = jnp.maximum(y, 0.0)
        sn_ref[...] = jnp.dot(hb, ws_ref[...], precision=_HIGH,
                              preferred_element_type=jnp.float32)
        p = jnp.dot(hb, wn_ref[...], precision=_HIGH,
                    preferred_element_type=jnp.float32)
        pn_ref[0] = p[:, :half]
        pn_ref[1] = p[:, half:]

    return pl.pallas_call(
        kern,
        grid=(n // _ROWBLK,),
        in_specs=[
            pl.BlockSpec((_ROWBLK, h), lambda i: (i, 0)),
            pl.BlockSpec((_NC, _ROWBLK, ph), lambda i: (0, i, 0)),
            pl.BlockSpec((_NC, _ROWBLK, _DEGW), lambda i: (0, i, 0)),
            pl.BlockSpec((1, h), lambda i: (0, 0)),
            pl.BlockSpec((1, h), lambda i: (0, 0)),
            pl.BlockSpec((1, h), lambda i: (0, 0)),
            pl.BlockSpec((h, wo), lambda i: (0, 0)),
            pl.BlockSpec((h, wo), lambda i: (0, 0)),
        ],
        out_specs=[
            pl.BlockSpec((_ROWBLK, wo), lambda i: (i, 0)),
            pl.BlockSpec((_NC, _ROWBLK, half), lambda i: (0, i, 0)),
        ],
        out_shape=[
            jax.ShapeDtypeStruct((n, wo), jnp.float32),
            jax.ShapeDtypeStruct((_NC, n, half), jnp.float32),
        ],
    )(s, parts, degparts, b.reshape(1, h), g.reshape(1, h), be.reshape(1, h),
      Wsn, Wnn)


def _tc_post(s, parts, degparts, b):
    n, w = s.shape
    ph = parts.shape[2]

    def kern(s_ref, parts_ref, deg_ref, b_ref, out_ref):
        out_ref[...] = s_ref[...] + _combine(parts_ref, deg_ref) + b_ref[...]

    return pl.pallas_call(
        kern,
        grid=(n // _ROWBLK,),
        in_specs=[
            pl.BlockSpec((_ROWBLK, w), lambda i: (i, 0)),
            pl.BlockSpec((_NC, _ROWBLK, ph), lambda i: (0, i, 0)),
            pl.BlockSpec((_NC, _ROWBLK, _DEGW), lambda i: (0, i, 0)),
            pl.BlockSpec((1, w), lambda i: (0, 0)),
        ],
        out_specs=pl.BlockSpec((_ROWBLK, w), lambda i: (i, 0)),
        out_shape=jax.ShapeDtypeStruct((n, w), jnp.float32),
    )(s, parts, degparts, b.reshape(1, w))


def kernel(x, edge_index, Ws0, Wn0, b0, g0, be0, Ws1, Wn1, b1, g1, be1,
           Ws2, Wn2, b2):
    src = edge_index[0].reshape(_NS, _NCHUNK, _CHUNK)
    dst = edge_index[1].reshape(_NS, _NCHUNK, _CHUNK)

    s0, p0 = _tc_pre(x, Ws0, Wn0)
    parts0, degp = _sc_agg(_H // 2, True, 4, 2)(p0, src, dst)
    s1, p1 = _tc_mid(s0, parts0, degp, b0, g0, be0, Ws1, Wn1)
    parts1 = _sc_agg(_H // 2, False, 6, 3)(p1, src, dst)
    s2, p2 = _tc_mid(s1, parts1, degp, b1, g1, be1, Ws2, Wn2)
    parts2 = _sc_agg(_C // 2, False, 8, 4)(p2, src, dst)
    return _tc_post(s2, parts2, degp, b2)
